# Initial kernel scaffold; baseline (speedup 1.0000x reference)
import jax
import jax.numpy as jnp
from jax.experimental import pallas as pl


def kernel(input_feat, pos_table, ln_gamma, ln_beta):
    raise NotImplementedError("write your pallas kernel here")


# speedup vs baseline: 3.5162x; 3.5162x over previous
import functools

import jax
import jax.numpy as jnp
from jax.experimental import pallas as pl


_EPS = 1e-5


def _ln_body(in_ref, pos_ref, gamma_ref, beta_ref, out_ref):
    x = in_ref[0] + pos_ref[...]
    mean = jnp.mean(x, axis=-1, keepdims=True)
    xc = x - mean
    var = jnp.mean(xc * xc, axis=-1, keepdims=True)
    rstd = jax.lax.rsqrt(var + _EPS)
    out_ref[0] = xc * rstd * gamma_ref[0] + beta_ref[0]


@functools.partial(jax.jit, static_argnames=("block_s",))
def _tc_kernel(input_feat, pos_table, ln_gamma, ln_beta, block_s=512):
    batch, seq, hidden = input_feat.shape
    grid = (seq // block_s, batch)
    return pl.pallas_call(
        _ln_body,
        grid=grid,
        in_specs=[
            pl.BlockSpec((1, block_s, hidden), lambda i, b: (b, i, 0)),
            pl.BlockSpec((block_s, hidden), lambda i, b: (i, 0)),
            pl.BlockSpec((1, hidden), lambda i, b: (0, 0)),
            pl.BlockSpec((1, hidden), lambda i, b: (0, 0)),
        ],
        out_specs=pl.BlockSpec((1, block_s, hidden), lambda i, b: (b, i, 0)),
        out_shape=jax.ShapeDtypeStruct(input_feat.shape, input_feat.dtype),
    )(input_feat, pos_table, ln_gamma.reshape(1, hidden), ln_beta.reshape(1, hidden))


def kernel(input_feat, pos_table, ln_gamma, ln_beta):
    seq = input_feat.shape[1]
    return _tc_kernel(input_feat, pos_table[:seq], ln_gamma, ln_beta)
